# baseline (speedup 1.0000x reference)
import functools

import jax
import jax.numpy as jnp
from jax import lax
from jax.experimental import pallas as pl
from jax.experimental.pallas import tpu as pltpu
from jax.experimental.pallas import tpu_sc as plsc

NC = 2
NS = 16
L = 16
NW = NC * NS

_EPS = 1e-6
_NEG_INF = float("-inf")
DIST_UNROLL = 8
NBUF = 3


def _vsqrt(s):
    s = jnp.maximum(s, 1e-30)
    seed = (plsc.bitcast(s, jnp.int32) >> 1) + 0x1FBD1DF5
    y = plsc.bitcast(seed, jnp.float32)
    for _ in range(4):
        y = 0.5 * (y + s / y)
    return y


@functools.lru_cache(maxsize=None)
def _build_amax(B, C):
    BLK = 256

    def body(p_ref, l_ref, o_ref):
        p = p_ref[...]
        lab = l_ref[...]
        cols = lax.broadcasted_iota(jnp.int32, (BLK, C), 1)
        masked = jnp.where(cols != lab[:, None], p, _NEG_INF)
        m = jnp.max(masked, axis=1, keepdims=True)
        hit = jnp.where(masked == m, cols, C)
        o_ref[...] = jnp.min(hit, axis=1).astype(jnp.int32)

    return pl.pallas_call(
        body,
        grid=(B // BLK,),
        in_specs=[pl.BlockSpec((BLK, C), lambda i: (i, 0)),
                  pl.BlockSpec((BLK,), lambda i: (i,))],
        out_specs=pl.BlockSpec((BLK,), lambda i: (i,)),
        out_shape=jax.ShapeDtypeStruct((B,), jnp.int32))


def _dist_loop(src, buf, rows, D, other):
    xblk, cblk = src
    skew = rows * 8

    def body(j, acc):
        base = j * DIST_UNROLL
        for k in range(DIST_UNROLL):
            col = (jnp.full((L,), base + k, jnp.int32) + skew) & (D - 1)
            xv = plsc.load_gather(xblk.at[buf], [rows, col])
            cv = plsc.load_gather(cblk.at[buf], [rows, col])
            t = xv - cv + _EPS
            acc = acc + t * t
        return acc

    return lax.fori_loop(0, D // DIST_UNROLL, body,
                         jnp.zeros((L,), jnp.float32))


def _sc_common(blocks):
    return dict(
        mesh=plsc.VectorSubcoreMesh(
            core_axis_name="c", subcore_axis_name="s",
            num_cores=NC, num_subcores=NS),
        compiler_params=pltpu.CompilerParams(
            use_tc_tiling_on_sc=False, needs_layout_passes=False),
    )


@functools.lru_cache(maxsize=None)
def _build_sc_ap(B, D, V):
    blocks = B // (NW * L)

    @functools.partial(
        pl.kernel,
        out_type=jax.ShapeDtypeStruct((B,), jnp.float32),
        scratch_types=[
            pltpu.VMEM((NBUF, L, D), jnp.float32),
            pltpu.VMEM((NBUF, L, D), jnp.float32),
            pltpu.VMEM((blocks * L,), jnp.int32),
            pltpu.VMEM((blocks * L,), jnp.float32),
        ] + [pltpu.SemaphoreType.DMA] * (2 * NBUF + 1),
        **_sc_common(blocks),
    )
    def sc_ap(x_hbm, labels_hbm, centers_hbm, out_hbm,
              xblk, posb, labv, aapv, *sems):
        wid = lax.axis_index("c") * NS + lax.axis_index("s")
        rows = lax.iota(jnp.int32, L)
        wbase = wid * (blocks * L)
        sx, spos, slab = sems[0:NBUF], sems[NBUF:2 * NBUF], sems[2 * NBUF]

        def start_block(b):
            buf = b % NBUF
            cx = pltpu.async_copy(
                x_hbm.at[pl.ds(wbase + b * L, L), :], xblk.at[buf], sx[buf])
            cpos = pltpu.async_copy(
                centers_hbm.at[labv.at[pl.ds(b * L, L)]], posb.at[buf],
                spos[buf])
            return cx, cpos

        pltpu.async_copy(labels_hbm.at[pl.ds(wbase, blocks * L)], labv,
                         slab).wait()
        cps = [None] * blocks
        for b in range(NBUF - 1):
            cps[b] = start_block(b)
        for b in range(blocks):
            if b + NBUF - 1 < blocks:
                cps[b + NBUF - 1] = start_block(b + NBUF - 1)
            for c in cps[b]:
                c.wait()
            aapv[pl.ds(b * L, L)] = _dist_loop((xblk, posb), b % NBUF,
                                               rows, D, None)
        pltpu.sync_copy(aapv, out_hbm.at[pl.ds(wbase, blocks * L)])

    return sc_ap


@functools.lru_cache(maxsize=None)
def _build_sc_an(B, D, V):
    blocks = B // (NW * L)

    @functools.partial(
        pl.kernel,
        out_type=jax.ShapeDtypeStruct((NW * L,), jnp.float32),
        scratch_types=[
            pltpu.VMEM((NBUF, L, D), jnp.float32),
            pltpu.VMEM((NBUF, L, D), jnp.float32),
            pltpu.VMEM((blocks * L,), jnp.int32),
            pltpu.VMEM((blocks * L,), jnp.float32),
            pltpu.VMEM((L,), jnp.float32),
        ] + [pltpu.SemaphoreType.DMA] * (2 * NBUF + 2),
        **_sc_common(blocks),
    )
    def sc_an(x_hbm, adv_hbm, aap_hbm, centers_hbm, out_hbm,
              xblk, negb, advv, aapv, partv, *sems):
        wid = lax.axis_index("c") * NS + lax.axis_index("s")
        rows = lax.iota(jnp.int32, L)
        wbase = wid * (blocks * L)
        sx, sneg = sems[0:NBUF], sems[NBUF:2 * NBUF]
        sadv, sap = sems[2 * NBUF], sems[2 * NBUF + 1]

        def start_block(b):
            buf = b % NBUF
            cx = pltpu.async_copy(
                x_hbm.at[pl.ds(wbase + b * L, L), :], xblk.at[buf], sx[buf])
            cneg = pltpu.async_copy(
                centers_hbm.at[advv.at[pl.ds(b * L, L)]], negb.at[buf],
                sneg[buf])
            return cx, cneg

        ca = pltpu.async_copy(adv_hbm.at[pl.ds(wbase, blocks * L)], advv,
                              sadv)
        cp = pltpu.async_copy(aap_hbm.at[pl.ds(wbase, blocks * L)], aapv,
                              sap)
        ca.wait()
        cps = [None] * blocks
        for b in range(NBUF - 1):
            cps[b] = start_block(b)
        cp.wait()

        part = jnp.zeros((L,), jnp.float32)
        for b in range(blocks):
            if b + NBUF - 1 < blocks:
                cps[b + NBUF - 1] = start_block(b + NBUF - 1)
            for c in cps[b]:
                c.wait()
            aan = _dist_loop((xblk, negb), b % NBUF, rows, D, None)
            aap = aapv[pl.ds(b * L, L)]
            part = part + jnp.maximum(_vsqrt(aap) - _vsqrt(aan) + 1.0, 0.0)

        partv[...] = part
        pltpu.sync_copy(partv, out_hbm.at[pl.ds(wid * L, L)])

    return sc_an


@functools.lru_cache(maxsize=None)
def _build_finish(B, P):
    def body(p_ref, o_ref):
        o_ref[...] = jnp.sum(p_ref[...], keepdims=True) * (1.0 / B)

    return pl.pallas_call(
        body, out_shape=jax.ShapeDtypeStruct((1,), jnp.float32))


def kernel(x, preds, labels, centers):
    B, D = x.shape
    C = preds.shape[1]
    V = centers.shape[0]
    labels = labels.astype(jnp.int32)
    adv = _build_amax(B, C)(preds, labels)
    aap = _build_sc_ap(B, D, V)(x, labels, centers)
    parts = _build_sc_an(B, D, V)(x, adv, aap, centers)
    return _build_finish(B, NW * L)(parts)[0]
